# baseline (speedup 1.0000x reference)
import functools

import jax
import jax.numpy as jnp
from jax import lax
from jax.experimental import pallas as pl
from jax.experimental.pallas import tpu as pltpu
from jax.experimental.pallas import tpu_sc as plsc

_N = 4 * 8192 * 2048
_NC = 2
_NS = 16
_NW = _NC * _NS
_PER_W = _N // _NW
_CHUNK = 16384
_ITERS = _PER_W // _CHUNK
_UNROLL = 8


def _sc_body(x_hbm, y_hbm, out_hbm, xv, yv, ov):
    c = lax.axis_index("c")
    s = lax.axis_index("s")
    wid = s * _NC + c
    base = wid * _PER_W

    def chunk_body(i, accs):
        acc_d, acc_y = accs
        off = base + i * _CHUNK
        pltpu.sync_copy(x_hbm.at[pl.ds(off, _CHUNK)], xv)
        pltpu.sync_copy(y_hbm.at[pl.ds(off, _CHUNK)], yv)

        def inner(k, accs2):
            ad, ay = accs2
            for j in range(_UNROLL):
                sl = pl.ds((k * _UNROLL + j) * 16, 16)
                xx = xv[sl]
                yy = yv[sl]
                d = xx - yy
                ad = ad + d * d
                ay = ay + yy * yy
            return (ad, ay)

        return lax.fori_loop(0, _CHUNK // (16 * _UNROLL), inner,
                             (acc_d, acc_y))

    zero = jnp.zeros((16,), jnp.float32)
    acc_d, acc_y = lax.fori_loop(0, _ITERS, chunk_body, (zero, zero))
    ov[pl.ds(0, 16)] = acc_d
    ov[pl.ds(16, 16)] = acc_y
    pltpu.sync_copy(ov, out_hbm.at[wid])


def kernel(x, y):
    xf = x.reshape(_N)
    yf = y.reshape(_N)
    mesh = plsc.VectorSubcoreMesh(core_axis_name="c", subcore_axis_name="s")
    partials = pl.kernel(
        _sc_body,
        mesh=mesh,
        out_type=jax.ShapeDtypeStruct((_NW, 32), jnp.float32),
        scratch_types=[
            pltpu.VMEM((_CHUNK,), jnp.float32),
            pltpu.VMEM((_CHUNK,), jnp.float32),
            pltpu.VMEM((32,), jnp.float32),
        ],
    )(xf, yf)
    sum_d = jnp.sum(partials[:, :16])
    sum_y = jnp.sum(partials[:, 16:])
    return sum_d / (2.0 * sum_y)


# speedup vs baseline: 1.5769x; 1.5769x over previous
import jax
import jax.numpy as jnp
from jax import lax
from jax.experimental import pallas as pl
from jax.experimental.pallas import tpu as pltpu
from jax.experimental.pallas import tpu_sc as plsc

_ROWS = 32768
_COLS = 2048

_SC_ROWS = 5120
_TC_ROWS = _ROWS - _SC_ROWS
_TC_BLOCK = 1024
_TC_GRID = _TC_ROWS // _TC_BLOCK

_NC = 2
_NS = 16
_NW = _NC * _NS
_SC_N = _SC_ROWS * _COLS
_PER_W = _SC_N // _NW
_CHUNK = 16384
_SC_ITERS = _PER_W // _CHUNK
_UNROLL = 8


def _tc_body(x_ref, y_ref, out_ref, acc_ref):
    i = pl.program_id(0)

    @pl.when(i == 0)
    def _init():
        acc_ref[0] = 0.0
        acc_ref[1] = 0.0

    x = x_ref[...]
    y = y_ref[...]
    d = x - y
    acc_ref[0] += jnp.sum(d * d)
    acc_ref[1] += jnp.sum(y * y)

    @pl.when(i == _TC_GRID - 1)
    def _finish():
        out_ref[0] = acc_ref[0]
        out_ref[1] = acc_ref[1]


def _sc_body(x_hbm, y_hbm, out_hbm, xv, yv, ov):
    c = lax.axis_index("c")
    s = lax.axis_index("s")
    wid = s * _NC + c
    base = _TC_ROWS * _COLS + wid * _PER_W

    def chunk_body(i, accs):
        acc_d, acc_y = accs
        off = base + i * _CHUNK
        pltpu.sync_copy(x_hbm.at[pl.ds(off, _CHUNK)], xv)
        pltpu.sync_copy(y_hbm.at[pl.ds(off, _CHUNK)], yv)

        def inner(k, accs2):
            ad, ay = accs2
            for j in range(_UNROLL):
                sl = pl.ds((k * _UNROLL + j) * 16, 16)
                xx = xv[sl]
                yy = yv[sl]
                d = xx - yy
                ad = ad + d * d
                ay = ay + yy * yy
            return (ad, ay)

        return lax.fori_loop(0, _CHUNK // (16 * _UNROLL), inner,
                             (acc_d, acc_y))

    zero = jnp.zeros((16,), jnp.float32)
    acc_d, acc_y = lax.fori_loop(0, _SC_ITERS, chunk_body, (zero, zero))
    ov[pl.ds(0, 16)] = acc_d
    ov[pl.ds(16, 16)] = acc_y
    pltpu.sync_copy(ov, out_hbm.at[wid])


def kernel(x, y):
    x2 = x.reshape(_ROWS, _COLS)
    y2 = y.reshape(_ROWS, _COLS)

    sc_x = x.reshape(_ROWS * _COLS)
    sc_y = y.reshape(_ROWS * _COLS)
    mesh = plsc.VectorSubcoreMesh(core_axis_name="c", subcore_axis_name="s")
    sc_partials = pl.kernel(
        _sc_body,
        mesh=mesh,
        out_type=jax.ShapeDtypeStruct((_NW, 32), jnp.float32),
        scratch_types=[
            pltpu.VMEM((_CHUNK,), jnp.float32),
            pltpu.VMEM((_CHUNK,), jnp.float32),
            pltpu.VMEM((32,), jnp.float32),
        ],
    )(sc_x, sc_y)

    tc_sums = pl.pallas_call(
        _tc_body,
        grid=(_TC_GRID,),
        in_specs=[
            pl.BlockSpec((_TC_BLOCK, _COLS), lambda i: (i, 0)),
            pl.BlockSpec((_TC_BLOCK, _COLS), lambda i: (i, 0)),
        ],
        out_specs=pl.BlockSpec(memory_space=pltpu.SMEM),
        out_shape=jax.ShapeDtypeStruct((2,), jnp.float32),
        scratch_shapes=[pltpu.SMEM((2,), jnp.float32)],
    )(x2, y2)

    sum_d = tc_sums[0] + jnp.sum(sc_partials[:, :16])
    sum_y = tc_sums[1] + jnp.sum(sc_partials[:, 16:])
    return sum_d / (2.0 * sum_y)


# speedup vs baseline: 4.8293x; 3.0625x over previous
import jax
import jax.numpy as jnp
from jax import lax
from jax.experimental import pallas as pl
from jax.experimental.pallas import tpu as pltpu
from jax.experimental.pallas import tpu_sc as plsc

_ROWS = 32768
_COLS = 2048

_SC_ROWS = 5120
_TC_ROWS = _ROWS - _SC_ROWS
_TC_BLOCK = 1024
_TC_GRID = _TC_ROWS // _TC_BLOCK

_NC = 2
_NS = 16
_NW = _NC * _NS
_ROWS_PER_W = _SC_ROWS // _NW
_STRIPE = 8
_SC_ITERS = _ROWS_PER_W // _STRIPE
_UNROLL = 4


def _tc_body(x_ref, y_ref, out_ref, acc_ref):
    i = pl.program_id(0)

    @pl.when(i == 0)
    def _init():
        acc_ref[0] = 0.0
        acc_ref[1] = 0.0

    x = x_ref[...]
    y = y_ref[...]
    d = x - y
    acc_ref[0] += jnp.sum(d * d)
    acc_ref[1] += jnp.sum(y * y)

    @pl.when(i == _TC_GRID - 1)
    def _finish():
        out_ref[0] = acc_ref[0]
        out_ref[1] = acc_ref[1]


def _sc_body(x_hbm, y_hbm, out_hbm, xv, yv, ov):
    c = lax.axis_index("c")
    s = lax.axis_index("s")
    wid = s * _NC + c
    base_row = _TC_ROWS + wid * _ROWS_PER_W

    def stripe_body(i, accs):
        acc_d, acc_y = accs
        row = base_row + i * _STRIPE
        pltpu.sync_copy(x_hbm.at[pl.ds(row, _STRIPE)], xv)
        pltpu.sync_copy(y_hbm.at[pl.ds(row, _STRIPE)], yv)

        def inner(k, accs2):
            ad, ay = accs2
            for j in range(_UNROLL):
                col = (k * _UNROLL + j) * 16
                for r in range(_STRIPE):
                    xx = xv[r, pl.ds(col, 16)]
                    yy = yv[r, pl.ds(col, 16)]
                    d = xx - yy
                    ad = ad + d * d
                    ay = ay + yy * yy
            return (ad, ay)

        return lax.fori_loop(0, _COLS // (16 * _UNROLL), inner,
                             (acc_d, acc_y))

    zero = jnp.zeros((16,), jnp.float32)
    acc_d, acc_y = lax.fori_loop(0, _SC_ITERS, stripe_body, (zero, zero))
    ov[pl.ds(0, 16)] = acc_d
    ov[pl.ds(16, 16)] = acc_y
    pltpu.sync_copy(ov, out_hbm.at[wid])


def kernel(x, y):
    x2 = x.reshape(_ROWS, _COLS)
    y2 = y.reshape(_ROWS, _COLS)

    mesh = plsc.VectorSubcoreMesh(core_axis_name="c", subcore_axis_name="s")
    sc_partials = pl.kernel(
        _sc_body,
        mesh=mesh,
        out_type=jax.ShapeDtypeStruct((_NW, 32), jnp.float32),
        scratch_types=[
            pltpu.VMEM((_STRIPE, _COLS), jnp.float32),
            pltpu.VMEM((_STRIPE, _COLS), jnp.float32),
            pltpu.VMEM((32,), jnp.float32),
        ],
        compiler_params=pltpu.CompilerParams(use_tc_tiling_on_sc=True),
    )(x2, y2)

    tc_sums = pl.pallas_call(
        _tc_body,
        grid=(_TC_GRID,),
        in_specs=[
            pl.BlockSpec((_TC_BLOCK, _COLS), lambda i: (i, 0)),
            pl.BlockSpec((_TC_BLOCK, _COLS), lambda i: (i, 0)),
        ],
        out_specs=pl.BlockSpec(memory_space=pltpu.SMEM),
        out_shape=jax.ShapeDtypeStruct((2,), jnp.float32),
        scratch_shapes=[pltpu.SMEM((2,), jnp.float32)],
    )(x2, y2)

    sum_d = tc_sums[0] + jnp.sum(sc_partials[:, :16])
    sum_y = tc_sums[1] + jnp.sum(sc_partials[:, 16:])
    return sum_d / (2.0 * sum_y)


# speedup vs baseline: 4.8623x; 1.0068x over previous
import jax
import jax.numpy as jnp
from jax import lax
from jax.experimental import pallas as pl
from jax.experimental.pallas import tpu as pltpu
from jax.experimental.pallas import tpu_sc as plsc

_ROWS = 32768
_COLS = 2048

_SC_ROWS = 2048
_TC_ROWS = _ROWS - _SC_ROWS
_TC_BLOCK = 1024
_TC_GRID = _TC_ROWS // _TC_BLOCK

_NC = 2
_NS = 16
_NW = _NC * _NS
_ROWS_PER_W = _SC_ROWS // _NW
_STRIPE = 8
_SC_ITERS = _ROWS_PER_W // _STRIPE
_UNROLL = 4


def _tc_body(x_ref, y_ref, out_ref, acc_ref):
    i = pl.program_id(0)

    @pl.when(i == 0)
    def _init():
        acc_ref[0] = 0.0
        acc_ref[1] = 0.0

    x = x_ref[...]
    y = y_ref[...]
    d = x - y
    acc_ref[0] += jnp.sum(d * d)
    acc_ref[1] += jnp.sum(y * y)

    @pl.when(i == _TC_GRID - 1)
    def _finish():
        out_ref[0] = acc_ref[0]
        out_ref[1] = acc_ref[1]


def _sc_body(x_hbm, y_hbm, out_hbm, xv, yv, ov):
    c = lax.axis_index("c")
    s = lax.axis_index("s")
    wid = s * _NC + c
    base_row = _TC_ROWS + wid * _ROWS_PER_W

    def stripe_body(i, accs):
        acc_d, acc_y = accs
        row = base_row + i * _STRIPE
        pltpu.sync_copy(x_hbm.at[pl.ds(row, _STRIPE)], xv)
        pltpu.sync_copy(y_hbm.at[pl.ds(row, _STRIPE)], yv)

        def inner(k, accs2):
            ad, ay = accs2
            for j in range(_UNROLL):
                col = (k * _UNROLL + j) * 16
                for r in range(_STRIPE):
                    xx = xv[r, pl.ds(col, 16)]
                    yy = yv[r, pl.ds(col, 16)]
                    d = xx - yy
                    ad = ad + d * d
                    ay = ay + yy * yy
            return (ad, ay)

        return lax.fori_loop(0, _COLS // (16 * _UNROLL), inner,
                             (acc_d, acc_y))

    zero = jnp.zeros((16,), jnp.float32)
    acc_d, acc_y = lax.fori_loop(0, _SC_ITERS, stripe_body, (zero, zero))
    ov[pl.ds(0, 16)] = acc_d
    ov[pl.ds(16, 16)] = acc_y
    pltpu.sync_copy(ov, out_hbm.at[wid])


def kernel(x, y):
    x2 = x.reshape(_ROWS, _COLS)
    y2 = y.reshape(_ROWS, _COLS)

    mesh = plsc.VectorSubcoreMesh(core_axis_name="c", subcore_axis_name="s")
    sc_partials = pl.kernel(
        _sc_body,
        mesh=mesh,
        out_type=jax.ShapeDtypeStruct((_NW, 32), jnp.float32),
        scratch_types=[
            pltpu.VMEM((_STRIPE, _COLS), jnp.float32),
            pltpu.VMEM((_STRIPE, _COLS), jnp.float32),
            pltpu.VMEM((32,), jnp.float32),
        ],
        compiler_params=pltpu.CompilerParams(use_tc_tiling_on_sc=True),
    )(x2, y2)

    tc_sums = pl.pallas_call(
        _tc_body,
        grid=(_TC_GRID,),
        in_specs=[
            pl.BlockSpec((_TC_BLOCK, _COLS), lambda i: (i, 0)),
            pl.BlockSpec((_TC_BLOCK, _COLS), lambda i: (i, 0)),
        ],
        out_specs=pl.BlockSpec(memory_space=pltpu.SMEM),
        out_shape=jax.ShapeDtypeStruct((2,), jnp.float32),
        scratch_shapes=[pltpu.SMEM((2,), jnp.float32)],
    )(x2, y2)

    sum_d = tc_sums[0] + jnp.sum(sc_partials[:, :16])
    sum_y = tc_sums[1] + jnp.sum(sc_partials[:, 16:])
    return sum_d / (2.0 * sum_y)
